# baseline (speedup 1.0000x reference)
import functools

import jax
import jax.numpy as jnp
from jax import lax
from jax.experimental import pallas as pl
from jax.experimental.pallas import tpu as pltpu
from jax.experimental.pallas import tpu_sc as plsc

_NW = 32
_RPD = 128


def _tbl_body(xyz_ref, feat_ref, ipad_ref, out_ref, *, nb_data, nd):
    i = pl.program_id(0)

    @pl.when(i < nb_data)
    def _():
        x = xyz_ref[0]
        f = feat_ref[0]
        w = x.shape[2]
        X = jnp.concatenate([x, f], axis=0)
        r = lax.dot_general(
            X, ipad_ref[...], (((0,), (0,)), ((), ())),
            precision=lax.Precision.HIGHEST,
            preferred_element_type=jnp.float32)
        out_ref[...] = r.reshape(8 * w, nd)

    @pl.when(i >= nb_data)
    def _():
        out_ref[...] = jnp.zeros_like(out_ref)


def _make_gather(nb, nd):
    per_w = nb // _NW
    ndma = per_w // _RPD

    @functools.partial(
        pl.kernel,
        mesh=plsc.VectorSubcoreMesh(core_axis_name="c", subcore_axis_name="s"),
        out_type=jax.ShapeDtypeStruct((nb, nd), jnp.float32),
        scratch_types=[
            pltpu.VMEM((ndma, _RPD), jnp.int32),
            pltpu.VMEM((_RPD, nd), jnp.float32),
            pltpu.SemaphoreType.DMA,
        ],
        compiler_params=pltpu.CompilerParams(use_tc_tiling_on_sc=False),
    )
    def gather_kernel(tbl_hbm, idx_hbm, out_hbm, idx_v, rows_v, sem):
        wid = lax.axis_index("s") * 2 + lax.axis_index("c")
        pltpu.sync_copy(idx_hbm.at[pl.ds(wid * ndma, ndma)], idx_v)
        base = wid * per_w

        def body(j, carry):
            pltpu.async_copy(tbl_hbm.at[idx_v.at[j]], rows_v, sem).wait()
            pltpu.sync_copy(rows_v, out_hbm.at[pl.ds(base + j * _RPD, _RPD)])
            return carry

        lax.fori_loop(0, ndma, body, 0)

    return gather_kernel


def _tc_body(g_ref, samp_ref, w1_ref, b1_ref, w2_ref, b2_ref, wl_ref, bl_ref,
             out_ref, *, mt, kk, nd, nh, nj):
    g = g_ref[...]
    g3 = g.reshape(mt, kk, nd)
    s = samp_ref[0].T
    xyzn = g3[:, :, 0:3] - s[:, None, :]
    x2 = xyzn.reshape(mt * kk, 3)
    hid = jnp.dot(x2, w1_ref[...], preferred_element_type=jnp.float32)
    hid = hid + b1_ref[...]
    hid = jnp.where(hid >= 0, hid, 0.1 * hid)
    wts = jnp.dot(hid, w2_ref[...], preferred_element_type=jnp.float32)
    wts = wts + b2_ref[...]
    wts = jnp.where(wts >= 0, wts, 0.1 * wts)
    w3 = wts.reshape(mt, kk, nj)
    acc = jax.lax.dot_general(
        w3, g3, (((1,), (1,)), ((0,), (0,))),
        preferred_element_type=jnp.float32)
    flat = acc.reshape(mt, nj * nd)
    o = jnp.dot(flat, wl_ref[...], preferred_element_type=jnp.float32)
    o = o + bl_ref[...]
    out_ref[...] = jnp.where(o >= 0, o, 0.1 * o).T


def kernel(xyz, features, sampled_xyz, knn_indices, valid_knn_mask,
           w1, b1, w2, b2, w_lin, b_lin):
    B, C, H, W = features.shape
    hh, ww = sampled_xyz.shape[2], sampled_xyz.shape[3]
    N = H * W
    M = hh * ww
    K = knn_indices.shape[2]
    CF = C + 3
    ND = 48
    NH = w1.shape[0]
    NJ = w2.shape[0]
    OC = w_lin.shape[0]

    TB = 8 * W
    npb = H // 8
    nb_data = B * npb
    ipad = jnp.pad(jnp.eye(CF, dtype=jnp.float32), ((0, 0), (0, ND - CF)))
    tbl = pl.pallas_call(
        functools.partial(_tbl_body, nb_data=nb_data, nd=ND),
        grid=(nb_data + 1,),
        in_specs=[
            pl.BlockSpec((1, 3, 8, W), lambda i: (jnp.minimum(i, nb_data - 1) // npb, 0,
                                                  jnp.minimum(i, nb_data - 1) % npb, 0)),
            pl.BlockSpec((1, C, 8, W), lambda i: (jnp.minimum(i, nb_data - 1) // npb, 0,
                                                  jnp.minimum(i, nb_data - 1) % npb, 0)),
            pl.BlockSpec((CF, ND), lambda i: (0, 0)),
        ],
        out_specs=pl.BlockSpec((TB, ND), lambda i: (i, 0)),
        out_shape=jax.ShapeDtypeStruct((B * N + TB, ND), jnp.float32),
    )(xyz, features, ipad)

    offs = (jnp.arange(B, dtype=jnp.int32) * N)[:, None, None]
    idx = knn_indices.astype(jnp.int32) + offs
    idx = jnp.where(valid_knn_mask, idx, B * N)
    idx2 = idx.reshape(-1, _RPD)

    g = _make_gather(B * M * K, ND)(tbl, idx2)

    sampf = sampled_xyz.reshape(B, 3, M)
    w1t = w1.T
    w2t = w2.T
    wlp = jnp.pad(w_lin.reshape(OC, NJ, CF), ((0, 0), (0, 0), (0, ND - CF)))
    wlp = wlp.reshape(OC, NJ * ND).T
    b1r = b1.reshape(1, NH)
    b2r = b2.reshape(1, NJ)
    blr = b_lin.reshape(1, OC)

    MT = 256
    tpb = M // MT
    grid = (B * tpb,)
    o = pl.pallas_call(
        functools.partial(_tc_body, mt=MT, kk=K, nd=ND, nh=NH, nj=NJ),
        grid=grid,
        in_specs=[
            pl.BlockSpec((MT * K, ND), lambda i: (i, 0)),
            pl.BlockSpec((1, 3, MT), lambda i: (i // tpb, 0, i % tpb)),
            pl.BlockSpec((3, NH), lambda i: (0, 0)),
            pl.BlockSpec((1, NH), lambda i: (0, 0)),
            pl.BlockSpec((NH, NJ), lambda i: (0, 0)),
            pl.BlockSpec((1, NJ), lambda i: (0, 0)),
            pl.BlockSpec((NJ * ND, OC), lambda i: (0, 0)),
            pl.BlockSpec((1, OC), lambda i: (0, 0)),
        ],
        out_specs=pl.BlockSpec((OC, MT), lambda i: (i // tpb, i % tpb)),
        out_shape=jax.ShapeDtypeStruct((B * OC, M), jnp.float32),
    )(g, sampf, w1t, b1r, w2t, b2r, wlp, blr)

    return o.reshape(B, OC, hh, ww)


# speedup vs baseline: 1.0484x; 1.0484x over previous
import functools

import jax
import jax.numpy as jnp
from jax import lax
from jax.experimental import pallas as pl
from jax.experimental.pallas import tpu as pltpu
from jax.experimental.pallas import tpu_sc as plsc

_NW = 32
_RPD = 128


def _tbl_body(xyz_ref, feat_ref, ipad_ref, out_ref, *, nb_data, nd):
    i = pl.program_id(0)

    @pl.when(i < nb_data)
    def _():
        x = xyz_ref[0]
        f = feat_ref[0]
        w = x.shape[2]
        X = jnp.concatenate([x, f], axis=0)
        xh = X.astype(jnp.bfloat16).astype(jnp.float32)
        xl = X - xh
        dims = (((0,), (0,)), ((), ()))
        r = lax.dot_general(xh, ipad_ref[...], dims,
                            preferred_element_type=jnp.float32)
        r = r + lax.dot_general(xl, ipad_ref[...], dims,
                                preferred_element_type=jnp.float32)
        out_ref[...] = r.reshape(8 * w, nd)

    @pl.when(i >= nb_data)
    def _():
        out_ref[...] = jnp.zeros_like(out_ref)


def _make_gather(nb, nd):
    per_w = nb // _NW
    ndma = per_w // _RPD

    @functools.partial(
        pl.kernel,
        mesh=plsc.VectorSubcoreMesh(core_axis_name="c", subcore_axis_name="s"),
        out_type=jax.ShapeDtypeStruct((nb, nd), jnp.float32),
        scratch_types=[
            pltpu.VMEM((ndma, _RPD), jnp.int32),
            pltpu.VMEM((_RPD, nd), jnp.float32),
            pltpu.SemaphoreType.DMA,
        ],
        compiler_params=pltpu.CompilerParams(use_tc_tiling_on_sc=False),
    )
    def gather_kernel(tbl_hbm, idx_hbm, out_hbm, idx_v, rows_v, sem):
        wid = lax.axis_index("s") * 2 + lax.axis_index("c")
        pltpu.sync_copy(idx_hbm.at[pl.ds(wid * ndma, ndma)], idx_v)
        base = wid * per_w

        def body(j, carry):
            pltpu.async_copy(tbl_hbm.at[idx_v.at[j]], rows_v, sem).wait()
            pltpu.sync_copy(rows_v, out_hbm.at[pl.ds(base + j * _RPD, _RPD)])
            return carry

        lax.fori_loop(0, ndma, body, 0)

    return gather_kernel


def _tc_body(g_ref, samp_ref, w1_ref, b1_ref, w2_ref, b2_ref, wl_ref, bl_ref,
             out_ref, *, mt, kk, nd, nh, nj):
    g = g_ref[...]
    g3 = g.reshape(mt, kk, nd)
    s = samp_ref[0].T
    xyzn = g3[:, :, 0:3] - s[:, None, :]
    x2 = xyzn.reshape(mt * kk, 3)
    hid = jnp.dot(x2, w1_ref[...], preferred_element_type=jnp.float32)
    hid = hid + b1_ref[...]
    hid = jnp.where(hid >= 0, hid, 0.1 * hid)
    wts = jnp.dot(hid, w2_ref[...], preferred_element_type=jnp.float32)
    wts = wts + b2_ref[...]
    wts = jnp.where(wts >= 0, wts, 0.1 * wts)
    w3 = wts.reshape(mt, kk, nj)
    acc = jax.lax.dot_general(
        w3, g3, (((1,), (1,)), ((0,), (0,))),
        preferred_element_type=jnp.float32)
    flat = acc.reshape(mt, nj * nd)
    o = jnp.dot(flat, wl_ref[...], preferred_element_type=jnp.float32)
    o = o + bl_ref[...]
    out_ref[...] = jnp.where(o >= 0, o, 0.1 * o).T


def kernel(xyz, features, sampled_xyz, knn_indices, valid_knn_mask,
           w1, b1, w2, b2, w_lin, b_lin):
    B, C, H, W = features.shape
    hh, ww = sampled_xyz.shape[2], sampled_xyz.shape[3]
    N = H * W
    M = hh * ww
    K = knn_indices.shape[2]
    CF = C + 3
    ND = 48
    NH = w1.shape[0]
    NJ = w2.shape[0]
    OC = w_lin.shape[0]

    TB = 8 * W
    npb = H // 8
    nb_data = B * npb
    ipad = jnp.pad(jnp.eye(CF, dtype=jnp.float32), ((0, 0), (0, ND - CF)))
    tbl = pl.pallas_call(
        functools.partial(_tbl_body, nb_data=nb_data, nd=ND),
        grid=(nb_data + 1,),
        in_specs=[
            pl.BlockSpec((1, 3, 8, W), lambda i: (jnp.minimum(i, nb_data - 1) // npb, 0,
                                                  jnp.minimum(i, nb_data - 1) % npb, 0)),
            pl.BlockSpec((1, C, 8, W), lambda i: (jnp.minimum(i, nb_data - 1) // npb, 0,
                                                  jnp.minimum(i, nb_data - 1) % npb, 0)),
            pl.BlockSpec((CF, ND), lambda i: (0, 0)),
        ],
        out_specs=pl.BlockSpec((TB, ND), lambda i: (i, 0)),
        out_shape=jax.ShapeDtypeStruct((B * N + TB, ND), jnp.float32),
    )(xyz, features, ipad)

    offs = (jnp.arange(B, dtype=jnp.int32) * N)[:, None, None]
    idx = knn_indices.astype(jnp.int32) + offs
    idx = jnp.where(valid_knn_mask, idx, B * N)
    idx2 = idx.reshape(-1, _RPD)

    g = _make_gather(B * M * K, ND)(tbl, idx2)

    sampf = sampled_xyz.reshape(B, 3, M)
    w1t = w1.T
    w2t = w2.T
    wlp = jnp.pad(w_lin.reshape(OC, NJ, CF), ((0, 0), (0, 0), (0, ND - CF)))
    wlp = wlp.reshape(OC, NJ * ND).T
    b1r = b1.reshape(1, NH)
    b2r = b2.reshape(1, NJ)
    blr = b_lin.reshape(1, OC)

    MT = 256
    tpb = M // MT
    grid = (B * tpb,)
    o = pl.pallas_call(
        functools.partial(_tc_body, mt=MT, kk=K, nd=ND, nh=NH, nj=NJ),
        grid=grid,
        in_specs=[
            pl.BlockSpec((MT * K, ND), lambda i: (i, 0)),
            pl.BlockSpec((1, 3, MT), lambda i: (i // tpb, 0, i % tpb)),
            pl.BlockSpec((3, NH), lambda i: (0, 0)),
            pl.BlockSpec((1, NH), lambda i: (0, 0)),
            pl.BlockSpec((NH, NJ), lambda i: (0, 0)),
            pl.BlockSpec((1, NJ), lambda i: (0, 0)),
            pl.BlockSpec((NJ * ND, OC), lambda i: (0, 0)),
            pl.BlockSpec((1, OC), lambda i: (0, 0)),
        ],
        out_specs=pl.BlockSpec((OC, MT), lambda i: (i // tpb, i % tpb)),
        out_shape=jax.ShapeDtypeStruct((B * OC, M), jnp.float32),
    )(g, sampf, w1t, b1r, w2t, b2r, wlp, blr)

    return o.reshape(B, OC, hh, ww)


# speedup vs baseline: 1.4850x; 1.4165x over previous
import functools

import jax
import jax.numpy as jnp
from jax import lax
from jax.experimental import pallas as pl
from jax.experimental.pallas import tpu as pltpu
from jax.experimental.pallas import tpu_sc as plsc

_NW = 32
_RPD = 128


def _tbl_body(xyz_ref, feat_ref, ipad_ref, out_ref, *, nb_data, nd):
    i = pl.program_id(0)

    @pl.when(i < nb_data)
    def _():
        x = xyz_ref[0]
        f = feat_ref[0]
        w = x.shape[2]
        X = jnp.concatenate([x, f], axis=0)
        xh = X.astype(jnp.bfloat16).astype(jnp.float32)
        xl = X - xh
        dims = (((0,), (0,)), ((), ()))
        r = lax.dot_general(xh, ipad_ref[...], dims,
                            preferred_element_type=jnp.float32)
        r = r + lax.dot_general(xl, ipad_ref[...], dims,
                                preferred_element_type=jnp.float32)
        out_ref[...] = r.reshape(8 * w, 128)

    @pl.when(i >= nb_data)
    def _():
        out_ref[...] = jnp.zeros_like(out_ref)


def _make_gather(nb, nd):
    per_w = nb // _NW
    ndma = per_w // _RPD

    @functools.partial(
        pl.kernel,
        mesh=plsc.VectorSubcoreMesh(core_axis_name="c", subcore_axis_name="s"),
        out_type=jax.ShapeDtypeStruct((nb, nd), jnp.float32),
        scratch_types=[
            pltpu.VMEM((ndma, _RPD), jnp.int32),
            pltpu.VMEM((_RPD, nd), jnp.float32),
            pltpu.SemaphoreType.DMA,
        ],
        compiler_params=pltpu.CompilerParams(use_tc_tiling_on_sc=False),
    )
    def gather_kernel(tbl_hbm, idx_hbm, out_hbm, idx_v, rows_v, sem):
        wid = lax.axis_index("s") * 2 + lax.axis_index("c")
        pltpu.sync_copy(idx_hbm.at[pl.ds(wid * ndma, ndma)], idx_v)
        base = wid * per_w

        def body(j, carry):
            pltpu.async_copy(tbl_hbm.at[idx_v.at[j]], rows_v, sem).wait()
            pltpu.sync_copy(rows_v, out_hbm.at[pl.ds(base + j * _RPD, _RPD)])
            return carry

        lax.fori_loop(0, ndma, body, 0)

    return gather_kernel


def _tc_body(g_ref, samp_ref, w1_ref, b1_ref, w2_ref, b2_ref, wl_ref, bl_ref,
             out_ref, *, mt, kk, nd, nh, nj):
    g = g_ref[:, 0:nd]
    g3 = g.reshape(mt, kk, nd)
    s = samp_ref[0].T
    xyzn = g3[:, :, 0:3] - s[:, None, :]
    x2 = xyzn.reshape(mt * kk, 3)
    hid = jnp.dot(x2, w1_ref[...], preferred_element_type=jnp.float32)
    hid = hid + b1_ref[...]
    hid = jnp.where(hid >= 0, hid, 0.1 * hid)
    wts = jnp.dot(hid, w2_ref[...], preferred_element_type=jnp.float32)
    wts = wts + b2_ref[...]
    wts = jnp.where(wts >= 0, wts, 0.1 * wts)
    w3 = wts.reshape(mt, kk, nj)
    acc = jax.lax.dot_general(
        w3, g3, (((1,), (1,)), ((0,), (0,))),
        preferred_element_type=jnp.float32)
    flat = acc.reshape(mt, nj * nd)
    o = jnp.dot(flat, wl_ref[...], preferred_element_type=jnp.float32)
    o = o + bl_ref[...]
    out_ref[...] = jnp.where(o >= 0, o, 0.1 * o).T


def kernel(xyz, features, sampled_xyz, knn_indices, valid_knn_mask,
           w1, b1, w2, b2, w_lin, b_lin):
    B, C, H, W = features.shape
    hh, ww = sampled_xyz.shape[2], sampled_xyz.shape[3]
    N = H * W
    M = hh * ww
    K = knn_indices.shape[2]
    CF = C + 3
    ND = 48
    NH = w1.shape[0]
    NJ = w2.shape[0]
    OC = w_lin.shape[0]

    TB = 8 * W
    npb = H // 8
    nb_data = B * npb
    ipad = jnp.pad(jnp.eye(CF, dtype=jnp.float32), ((0, 0), (0, 128 - CF)))
    tbl = pl.pallas_call(
        functools.partial(_tbl_body, nb_data=nb_data, nd=ND),
        grid=(nb_data + 1,),
        in_specs=[
            pl.BlockSpec((1, 3, 8, W), lambda i: (jnp.minimum(i, nb_data - 1) // npb, 0,
                                                  jnp.minimum(i, nb_data - 1) % npb, 0)),
            pl.BlockSpec((1, C, 8, W), lambda i: (jnp.minimum(i, nb_data - 1) // npb, 0,
                                                  jnp.minimum(i, nb_data - 1) % npb, 0)),
            pl.BlockSpec((CF, 128), lambda i: (0, 0)),
        ],
        out_specs=pl.BlockSpec((TB, 128), lambda i: (i, 0)),
        out_shape=jax.ShapeDtypeStruct((B * N + TB, 128), jnp.float32),
    )(xyz, features, ipad)

    offs = (jnp.arange(B, dtype=jnp.int32) * N)[:, None, None]
    idx = knn_indices.astype(jnp.int32) + offs
    idx = jnp.where(valid_knn_mask, idx, B * N)
    idx2 = idx.reshape(-1, _RPD)

    g = _make_gather(B * M * K, 128)(tbl, idx2)

    sampf = sampled_xyz.reshape(B, 3, M)
    w1t = w1.T
    w2t = w2.T
    wlp = jnp.pad(w_lin.reshape(OC, NJ, CF), ((0, 0), (0, 0), (0, ND - CF)))
    wlp = wlp.reshape(OC, NJ * ND).T
    b1r = b1.reshape(1, NH)
    b2r = b2.reshape(1, NJ)
    blr = b_lin.reshape(1, OC)

    MT = 256
    tpb = M // MT
    grid = (B * tpb,)
    o = pl.pallas_call(
        functools.partial(_tc_body, mt=MT, kk=K, nd=ND, nh=NH, nj=NJ),
        grid=grid,
        in_specs=[
            pl.BlockSpec((MT * K, 128), lambda i: (i, 0)),
            pl.BlockSpec((1, 3, MT), lambda i: (i // tpb, 0, i % tpb)),
            pl.BlockSpec((3, NH), lambda i: (0, 0)),
            pl.BlockSpec((1, NH), lambda i: (0, 0)),
            pl.BlockSpec((NH, NJ), lambda i: (0, 0)),
            pl.BlockSpec((1, NJ), lambda i: (0, 0)),
            pl.BlockSpec((NJ * ND, OC), lambda i: (0, 0)),
            pl.BlockSpec((1, OC), lambda i: (0, 0)),
        ],
        out_specs=pl.BlockSpec((OC, MT), lambda i: (i // tpb, i % tpb)),
        out_shape=jax.ShapeDtypeStruct((B * OC, M), jnp.float32),
    )(g, sampf, w1t, b1r, w2t, b2r, wlp, blr)

    return o.reshape(B, OC, hh, ww)


# speedup vs baseline: 1.6878x; 1.1366x over previous
import functools

import jax
import jax.numpy as jnp
from jax import lax
from jax.experimental import pallas as pl
from jax.experimental.pallas import tpu as pltpu
from jax.experimental.pallas import tpu_sc as plsc

_NW = 32
_RPD = 128


def _tbl_body(xyz_ref, feat_ref, ipad_ref, out_ref, *, nb_data, nd):
    i = pl.program_id(0)

    @pl.when(i < nb_data)
    def _():
        x = xyz_ref[0]
        f = feat_ref[0]
        w = x.shape[2]
        X = jnp.concatenate([x, f], axis=0)
        xh = X.astype(jnp.bfloat16).astype(jnp.float32)
        xl = X - xh
        dims = (((0,), (0,)), ((), ()))
        r = lax.dot_general(xh, ipad_ref[...], dims,
                            preferred_element_type=jnp.float32)
        r = r + lax.dot_general(xl, ipad_ref[...], dims,
                                preferred_element_type=jnp.float32)
        out_ref[...] = r.reshape(8 * w, 128)

    @pl.when(i >= nb_data)
    def _():
        out_ref[...] = jnp.zeros_like(out_ref)


def _make_gather(nb, nd):
    per_w = nb // _NW
    ndma = per_w // _RPD

    @functools.partial(
        pl.kernel,
        mesh=plsc.VectorSubcoreMesh(core_axis_name="c", subcore_axis_name="s"),
        out_type=jax.ShapeDtypeStruct((nb, nd), jnp.float32),
        scratch_types=[
            pltpu.VMEM((ndma, _RPD), jnp.int32),
            pltpu.VMEM((_RPD, nd), jnp.float32),
            pltpu.SemaphoreType.DMA,
        ],
        compiler_params=pltpu.CompilerParams(use_tc_tiling_on_sc=False),
    )
    def gather_kernel(tbl_hbm, idx_hbm, out_hbm, idx_v, rows_v, sem):
        wid = lax.axis_index("s") * 2 + lax.axis_index("c")
        pltpu.sync_copy(idx_hbm.at[pl.ds(wid * ndma, ndma)], idx_v)
        base = wid * per_w

        def body(j, carry):
            pltpu.async_copy(tbl_hbm.at[idx_v.at[j]], rows_v, sem).wait()
            pltpu.sync_copy(rows_v, out_hbm.at[pl.ds(base + j * _RPD, _RPD)])
            return carry

        lax.fori_loop(0, ndma, body, 0)

    return gather_kernel


def _tc_body(g_ref, samp_ref, w1_ref, b1_ref, w2_ref, b2_ref, wl_ref, bl_ref,
             out_ref, *, mt, kk, nd, nh, nj):
    g = g_ref[:, 0:nd]
    g3 = g.reshape(mt, kk, nd)
    s = samp_ref[0].T
    xyzn = g3[:, :, 0:3] - s[:, None, :]
    x2 = xyzn.reshape(mt * kk, 3)
    hid = jnp.dot(x2, w1_ref[...], preferred_element_type=jnp.float32)
    hid = hid + b1_ref[...]
    hid = jnp.where(hid >= 0, hid, 0.1 * hid)
    wts = jnp.dot(hid, w2_ref[...], preferred_element_type=jnp.float32)
    wts = wts + b2_ref[...]
    wts = jnp.where(wts >= 0, wts, 0.1 * wts)
    w3 = wts.reshape(mt, kk, nj)
    acc = jax.lax.dot_general(
        w3, g3, (((1,), (1,)), ((0,), (0,))),
        preferred_element_type=jnp.float32)
    flat = acc.reshape(mt, nj * nd)
    o = jnp.dot(flat, wl_ref[...], preferred_element_type=jnp.float32)
    o = o + bl_ref[...]
    out_ref[...] = jnp.where(o >= 0, o, 0.1 * o).T


def kernel(xyz, features, sampled_xyz, knn_indices, valid_knn_mask,
           w1, b1, w2, b2, w_lin, b_lin):
    B, C, H, W = features.shape
    hh, ww = sampled_xyz.shape[2], sampled_xyz.shape[3]
    N = H * W
    M = hh * ww
    K = knn_indices.shape[2]
    CF = C + 3
    ND = 48
    NH = w1.shape[0]
    NJ = w2.shape[0]
    OC = w_lin.shape[0]

    TB = 8 * W
    npb = H // 8
    nb_data = B * npb
    ipad = jnp.pad(jnp.eye(CF, dtype=jnp.float32), ((0, 0), (0, 128 - CF)))
    tbl = pl.pallas_call(
        functools.partial(_tbl_body, nb_data=nb_data, nd=ND),
        grid=(nb_data + 1,),
        in_specs=[
            pl.BlockSpec((1, 3, 8, W), lambda i: (jnp.minimum(i, nb_data - 1) // npb, 0,
                                                  jnp.minimum(i, nb_data - 1) % npb, 0)),
            pl.BlockSpec((1, C, 8, W), lambda i: (jnp.minimum(i, nb_data - 1) // npb, 0,
                                                  jnp.minimum(i, nb_data - 1) % npb, 0)),
            pl.BlockSpec((CF, 128), lambda i: (0, 0)),
        ],
        out_specs=pl.BlockSpec((TB, 128), lambda i: (i, 0)),
        out_shape=jax.ShapeDtypeStruct((B * N + TB, 128), jnp.float32),
    )(xyz, features, ipad)

    offs = (jnp.arange(B, dtype=jnp.int32) * N)[:, None, None]
    idx = knn_indices.astype(jnp.int32) + offs
    idx = jnp.where(valid_knn_mask, idx, B * N)
    idx2 = idx.reshape(-1, _RPD)

    sampf = sampled_xyz.reshape(B, 3, M)
    w1t = w1.T
    w2t = w2.T
    wlp = jnp.pad(w_lin.reshape(OC, NJ, CF), ((0, 0), (0, 0), (0, ND - CF)))
    wlp = wlp.reshape(OC, NJ * ND).T
    b1r = b1.reshape(1, NH)
    b2r = b2.reshape(1, NJ)
    blr = b_lin.reshape(1, OC)

    MT = 256
    tpb = M // MT
    nrows_h = idx2.shape[0] // B

    gather_fn = _make_gather(M * K, 128)
    gs = [gather_fn(tbl, idx2[b * nrows_h:(b + 1) * nrows_h]) for b in range(B)]

    compute = pl.pallas_call(
        functools.partial(_tc_body, mt=MT, kk=K, nd=ND, nh=NH, nj=NJ),
        grid=(tpb,),
        in_specs=[
            pl.BlockSpec((MT * K, 128), lambda i: (i, 0)),
            pl.BlockSpec((1, 3, MT), lambda i: (0, 0, i)),
            pl.BlockSpec((3, NH), lambda i: (0, 0)),
            pl.BlockSpec((1, NH), lambda i: (0, 0)),
            pl.BlockSpec((NH, NJ), lambda i: (0, 0)),
            pl.BlockSpec((1, NJ), lambda i: (0, 0)),
            pl.BlockSpec((NJ * ND, OC), lambda i: (0, 0)),
            pl.BlockSpec((1, OC), lambda i: (0, 0)),
        ],
        out_specs=pl.BlockSpec((OC, MT), lambda i: (0, i)),
        out_shape=jax.ShapeDtypeStruct((OC, M), jnp.float32),
    )
    os_ = [compute(gs[b], sampf[b:b + 1], w1t, b1r, w2t, b2r, wlp, blr)
           for b in range(B)]
    o = jnp.stack(os_)

    return o.reshape(B, OC, hh, ww)


# speedup vs baseline: 1.7919x; 1.0617x over previous
import functools

import jax
import jax.numpy as jnp
from jax import lax
from jax.experimental import pallas as pl
from jax.experimental.pallas import tpu as pltpu
from jax.experimental.pallas import tpu_sc as plsc

_NW = 32
_RPD = 128


def _tbl_body(xyz_ref, feat_ref, ipad_ref, out_ref, *, nb_data, nd):
    i = pl.program_id(0)

    @pl.when(i < nb_data)
    def _():
        x = xyz_ref[0]
        f = feat_ref[0]
        w = x.shape[2]
        X = jnp.concatenate([x, f], axis=0)
        xh = X.astype(jnp.bfloat16).astype(jnp.float32)
        xl = X - xh
        dims = (((0,), (0,)), ((), ()))
        r = lax.dot_general(xh, ipad_ref[...], dims,
                            preferred_element_type=jnp.float32)
        r = r + lax.dot_general(xl, ipad_ref[...], dims,
                                preferred_element_type=jnp.float32)
        out_ref[...] = r.reshape(8 * w, 128)

    @pl.when(i >= nb_data)
    def _():
        out_ref[...] = jnp.zeros_like(out_ref)


def _make_gather(nb, nd):
    per_w = nb // _NW
    ndma = per_w // _RPD

    @functools.partial(
        pl.kernel,
        mesh=plsc.VectorSubcoreMesh(core_axis_name="c", subcore_axis_name="s"),
        out_type=jax.ShapeDtypeStruct((nb, nd), jnp.float32),
        scratch_types=[
            pltpu.VMEM((ndma, _RPD), jnp.int32),
            pltpu.VMEM((_RPD, nd), jnp.float32),
            pltpu.SemaphoreType.DMA,
        ],
        compiler_params=pltpu.CompilerParams(use_tc_tiling_on_sc=False),
    )
    def gather_kernel(tbl_hbm, idx_hbm, out_hbm, idx_v, rows_v, sem):
        wid = lax.axis_index("s") * 2 + lax.axis_index("c")
        pltpu.sync_copy(idx_hbm.at[pl.ds(wid * ndma, ndma)], idx_v)
        base = wid * per_w

        def body(j, carry):
            pltpu.async_copy(tbl_hbm.at[idx_v.at[j]], rows_v, sem).wait()
            pltpu.sync_copy(rows_v, out_hbm.at[pl.ds(base + j * _RPD, _RPD)])
            return carry

        lax.fori_loop(0, ndma, body, 0)

    return gather_kernel


def _tc_body(g_ref, samp_ref, w1_ref, b1_ref, w2_ref, b2_ref, wl_ref, bl_ref,
             out_ref, *, mt, kk, nd, nh, nj):
    g = g_ref[:, 0:nd]
    g3 = g.reshape(mt, kk, nd)
    s = samp_ref[0].T
    xyzn = g3[:, :, 0:3] - s[:, None, :]
    x2 = xyzn.reshape(mt * kk, 3)
    hid = jnp.dot(x2, w1_ref[...], preferred_element_type=jnp.float32)
    hid = hid + b1_ref[...]
    hid = jnp.where(hid >= 0, hid, 0.1 * hid)
    wts = jnp.dot(hid, w2_ref[...], preferred_element_type=jnp.float32)
    wts = wts + b2_ref[...]
    wts = jnp.where(wts >= 0, wts, 0.1 * wts)
    w3 = wts.reshape(mt, kk, nj)
    acc = jax.lax.dot_general(
        w3, g3, (((1,), (1,)), ((0,), (0,))),
        preferred_element_type=jnp.float32)
    flat = acc.reshape(mt, nj * nd)
    o = jnp.dot(flat, wl_ref[...], preferred_element_type=jnp.float32)
    o = o + bl_ref[...]
    out_ref[...] = jnp.where(o >= 0, o, 0.1 * o).T


def kernel(xyz, features, sampled_xyz, knn_indices, valid_knn_mask,
           w1, b1, w2, b2, w_lin, b_lin):
    B, C, H, W = features.shape
    hh, ww = sampled_xyz.shape[2], sampled_xyz.shape[3]
    N = H * W
    M = hh * ww
    K = knn_indices.shape[2]
    CF = C + 3
    ND = 48
    NH = w1.shape[0]
    NJ = w2.shape[0]
    OC = w_lin.shape[0]

    TB = 8 * W
    npb = H // 8
    nb_data = B * npb
    ipad = jnp.pad(jnp.eye(CF, dtype=jnp.float32), ((0, 0), (0, 128 - CF)))
    tbl = pl.pallas_call(
        functools.partial(_tbl_body, nb_data=nb_data, nd=ND),
        grid=(nb_data + 1,),
        in_specs=[
            pl.BlockSpec((1, 3, 8, W), lambda i: (jnp.minimum(i, nb_data - 1) // npb, 0,
                                                  jnp.minimum(i, nb_data - 1) % npb, 0)),
            pl.BlockSpec((1, C, 8, W), lambda i: (jnp.minimum(i, nb_data - 1) // npb, 0,
                                                  jnp.minimum(i, nb_data - 1) % npb, 0)),
            pl.BlockSpec((CF, 128), lambda i: (0, 0)),
        ],
        out_specs=pl.BlockSpec((TB, 128), lambda i: (i, 0)),
        out_shape=jax.ShapeDtypeStruct((B * N + TB, 128), jnp.float32),
    )(xyz, features, ipad)

    offs = (jnp.arange(B, dtype=jnp.int32) * N)[:, None, None]
    idx = knn_indices.astype(jnp.int32) + offs
    idx = jnp.where(valid_knn_mask, idx, B * N)
    idx2 = idx.reshape(-1, _RPD)

    w1t = w1.T
    w2t = w2.T
    wlp = jnp.pad(w_lin.reshape(OC, NJ, CF), ((0, 0), (0, 0), (0, ND - CF)))
    wlp = wlp.reshape(OC, NJ * ND).T
    b1r = b1.reshape(1, NH)
    b2r = b2.reshape(1, NJ)
    blr = b_lin.reshape(1, OC)

    MT = 256
    NCH = 2 * B
    MC = M // (NCH // B)
    tpb = MC // MT
    nrows_h = idx2.shape[0] // NCH

    gather_fn = _make_gather(MC * K, 128)
    gs = [gather_fn(tbl, idx2[c * nrows_h:(c + 1) * nrows_h]) for c in range(NCH)]
    samp3 = sampled_xyz.reshape(B, 3, M)

    compute = pl.pallas_call(
        functools.partial(_tc_body, mt=MT, kk=K, nd=ND, nh=NH, nj=NJ),
        grid=(tpb,),
        in_specs=[
            pl.BlockSpec((MT * K, 128), lambda i: (i, 0)),
            pl.BlockSpec((1, 3, MT), lambda i: (0, 0, i)),
            pl.BlockSpec((3, NH), lambda i: (0, 0)),
            pl.BlockSpec((1, NH), lambda i: (0, 0)),
            pl.BlockSpec((NH, NJ), lambda i: (0, 0)),
            pl.BlockSpec((1, NJ), lambda i: (0, 0)),
            pl.BlockSpec((NJ * ND, OC), lambda i: (0, 0)),
            pl.BlockSpec((1, OC), lambda i: (0, 0)),
        ],
        out_specs=pl.BlockSpec((OC, MT), lambda i: (0, i)),
        out_shape=jax.ShapeDtypeStruct((OC, MC), jnp.float32),
    )
    os_ = [compute(gs[c],
                   samp3[c // 2:c // 2 + 1, :, (c % 2) * MC:(c % 2) * MC + MC],
                   w1t, b1r, w2t, b2r, wlp, blr)
           for c in range(NCH)]
    o = jnp.stack(os_)

    return o.reshape(B, 2, OC, MC).transpose(0, 2, 1, 3).reshape(B, OC, hh, ww)
